# Initial kernel scaffold; baseline (speedup 1.0000x reference)
import jax
import jax.numpy as jnp
from jax.experimental import pallas as pl


def kernel(x, edge_index, edge_values, W):
    raise NotImplementedError("write your pallas kernel here")


# speedup vs baseline: 2.5615x; 2.5615x over previous
import functools

import jax
import jax.numpy as jnp
from jax import lax
from jax.experimental import pallas as pl
from jax.experimental.pallas import tpu as pltpu
from jax.experimental.pallas import tpu_sc as plsc

N_NODES = 10000
N_EDGES = 160000
D_IN = 256
D_OUT = 256
D_HALF = D_OUT // 2

NUM_CORES = 2
NUM_SUBCORES = 16
LANES = 16

EPT = N_EDGES // NUM_SUBCORES
EC = 128
EPT_PAD = ((EPT + EC - 1) // EC) * EC
N_PAD = 10240
NPASS = 4
ROWS_PASS = N_PAD // NPASS
RPT = ROWS_PASS // NUM_SUBCORES
WB = 160


def _matmul_body(x_ref, w_ref, o_ref):
    o_ref[0, :, :] = jnp.dot(x_ref[...], w_ref[...],
                             preferred_element_type=jnp.float32)


def _matmul_halves(x, W):
    BR = 1000
    grid = (N_NODES // BR, NUM_CORES)
    return pl.pallas_call(
        _matmul_body,
        grid=grid,
        in_specs=[
            pl.BlockSpec((BR, D_IN), lambda i, j: (i, 0)),
            pl.BlockSpec((D_IN, D_HALF), lambda i, j: (0, j)),
        ],
        out_specs=pl.BlockSpec((1, BR, D_HALF), lambda i, j: (j, i, 0)),
        out_shape=jax.ShapeDtypeStruct((NUM_CORES, N_NODES, D_HALF),
                                       jnp.float32),
    )(x, W)


def _sc_body(pre_hbm, col_hbm, row_hbm, val_hbm, out_hbm,
             colt, rowt, valt, colc, rowc, valc,
             cbuf, rbuf, msgs, zbuf, acc, sem):
    cid = lax.axis_index("c")
    sid = lax.axis_index("s")
    ebase = sid * EPT
    ngrp = EPT // LANES
    zeros16i = jnp.zeros((LANES,), jnp.int32)
    zeros16f = jnp.zeros((LANES,), jnp.float32)

    pltpu.sync_copy(col_hbm.at[pl.ds(ebase, EPT)], colt)
    pltpu.sync_copy(row_hbm.at[pl.ds(ebase, EPT)], rowt)
    pltpu.sync_copy(val_hbm.at[pl.ds(ebase, EPT)], valt)

    def prefill(g, _):
        colc[pl.ds(g * LANES, LANES)] = zeros16i
        rowc[pl.ds(g * LANES, LANES)] = zeros16i
        return 0
    lax.fori_loop(0, EPT_PAD // LANES, prefill, 0)

    coff = cid * N_NODES

    for p in range(NPASS):
        lo = p * ROWS_PASS

        def zfill(r, _):
            for j in range(D_HALF // LANES):
                zbuf[r, pl.ds(j * LANES, LANES)] = zeros16f
            return 0
        lax.fori_loop(0, WB, zfill, 0)
        for b in range(RPT // WB):
            pltpu.sync_copy(
                zbuf, acc.at[pl.ds(sid * RPT + b * WB, WB), :])

        def vfill(g, _):
            valc[pl.ds(g * LANES, LANES)] = zeros16f
            return 0
        lax.fori_loop(0, EPT_PAD // LANES, vfill, 0)

        def compact(g, cnt):
            sl = pl.ds(g * LANES, LANES)
            rows = rowt[sl]
            mask = (rows >= lo) & (rows < lo + ROWS_PASS)
            plsc.store_compressed(colc.at[pl.ds(cnt, LANES)],
                                  colt[sl] + coff, mask=mask)
            plsc.store_compressed(rowc.at[pl.ds(cnt, LANES)],
                                  rows - lo, mask=mask)
            plsc.store_compressed(valc.at[pl.ds(cnt, LANES)],
                                  valt[sl], mask=mask)
            return cnt + plsc.all_reduce_population_count(mask)[0]
        cnt = lax.fori_loop(0, ngrp, compact, 0)
        plsc.subcore_barrier()

        nchunk = (cnt + EC - 1) // EC

        def chunk(c, _):
            base = c * EC
            for k in range(EC // LANES):
                sl = pl.ds(base + k * LANES, LANES)
                dl = pl.ds(k * LANES, LANES)
                cbuf[dl] = colc[sl]
                rbuf[dl] = rowc[sl]
            pltpu.async_copy(pre_hbm.at[cbuf], msgs, sem).wait()

            def scale(g, _):
                vvals = valc[pl.ds(base + g * LANES, LANES)]
                for l in range(LANES):
                    v = vvals[l]
                    e = g * LANES + l
                    for j in range(D_HALF // LANES):
                        fl = pl.ds(j * LANES, LANES)
                        msgs[e, fl] = msgs[e, fl] * v
                return 0
            lax.fori_loop(0, EC // LANES, scale, 0)

            pltpu.sync_copy(msgs, acc.at[rbuf], add=True)
            return 0
        lax.fori_loop(0, nchunk, chunk, 0)
        plsc.subcore_barrier()

        for b in range(RPT // WB):
            r0 = sid * RPT + b * WB
            pltpu.sync_copy(acc.at[pl.ds(r0, WB), :], zbuf)

            def relu_row(r, _):
                for j in range(D_HALF // LANES):
                    fl = pl.ds(j * LANES, LANES)
                    zbuf[r, fl] = jnp.maximum(zbuf[r, fl], 0.0)
                return 0
            lax.fori_loop(0, WB, relu_row, 0)
            pltpu.sync_copy(
                zbuf, out_hbm.at[cid, pl.ds(lo + r0, WB), :])


_sc_spmm = functools.partial(
    pl.kernel,
    mesh=plsc.VectorSubcoreMesh(core_axis_name="c", subcore_axis_name="s"),
    compiler_params=pltpu.CompilerParams(needs_layout_passes=False),
    out_type=jax.ShapeDtypeStruct((NUM_CORES, N_PAD, D_HALF), jnp.float32),
    scratch_types=[
        pltpu.VMEM((EPT,), jnp.int32),
        pltpu.VMEM((EPT,), jnp.int32),
        pltpu.VMEM((EPT,), jnp.float32),
        pltpu.VMEM((EPT_PAD,), jnp.int32),
        pltpu.VMEM((EPT_PAD,), jnp.int32),
        pltpu.VMEM((EPT_PAD,), jnp.float32),
        pltpu.VMEM((EC,), jnp.int32),
        pltpu.VMEM((EC,), jnp.int32),
        pltpu.VMEM((EC, D_HALF), jnp.float32),
        pltpu.VMEM((WB, D_HALF), jnp.float32),
        pltpu.VMEM_SHARED((ROWS_PASS, D_HALF), jnp.float32),
        pltpu.SemaphoreType.DMA,
    ],
)(_sc_body)


def kernel(x, edge_index, edge_values, W):
    row = edge_index[0].astype(jnp.int32)
    col = edge_index[1].astype(jnp.int32)
    pre = _matmul_halves(x, W).reshape(NUM_CORES * N_NODES, D_HALF)
    out2 = _sc_spmm(pre, col, row, edge_values)
    return jnp.concatenate([out2[0, :N_NODES], out2[1, :N_NODES]], axis=1)


# speedup vs baseline: 3.0673x; 1.1974x over previous
import functools

import jax
import jax.numpy as jnp
from jax import lax
from jax.experimental import pallas as pl
from jax.experimental.pallas import tpu as pltpu
from jax.experimental.pallas import tpu_sc as plsc

N_NODES = 10000
N_EDGES = 160000
D_IN = 256
D_OUT = 256
D_HALF = D_OUT // 2

NUM_CORES = 2
NUM_SUBCORES = 16
LANES = 16

EPT = N_EDGES // NUM_SUBCORES
EC = 128
EPT_PAD = ((EPT + EC - 1) // EC) * EC
N_PAD = 10240
NPASS = 4
ROWS_PASS = N_PAD // NPASS
RPT = ROWS_PASS // NUM_SUBCORES
WB = 80


def _matmul_body(x_ref, w_ref, o_ref):
    o_ref[0, :, :] = jnp.dot(x_ref[...], w_ref[...],
                             preferred_element_type=jnp.float32)


def _matmul_halves(x, W):
    BR = 1000
    grid = (N_NODES // BR, NUM_CORES)
    return pl.pallas_call(
        _matmul_body,
        grid=grid,
        in_specs=[
            pl.BlockSpec((BR, D_IN), lambda i, j: (i, 0)),
            pl.BlockSpec((D_IN, D_HALF), lambda i, j: (0, j)),
        ],
        out_specs=pl.BlockSpec((1, BR, D_HALF), lambda i, j: (j, i, 0)),
        out_shape=jax.ShapeDtypeStruct((NUM_CORES, N_NODES, D_HALF),
                                       jnp.float32),
    )(x, W)


def _sc_body(pre_hbm, col_hbm, row_hbm, val_hbm, out_hbm,
             colt, rowt, valt, colc, rowc, valc,
             cbuf, rbuf, msgs, cbuf2, rbuf2, msgs2, zbuf, acc, gsem, ssem):
    cid = lax.axis_index("c")
    sid = lax.axis_index("s")
    ebase = sid * EPT
    ngrp = EPT // LANES
    zeros16i = jnp.zeros((LANES,), jnp.int32)
    zeros16f = jnp.zeros((LANES,), jnp.float32)

    pltpu.sync_copy(col_hbm.at[pl.ds(ebase, EPT)], colt)
    pltpu.sync_copy(row_hbm.at[pl.ds(ebase, EPT)], rowt)
    pltpu.sync_copy(val_hbm.at[pl.ds(ebase, EPT)], valt)

    def prefill(g, _):
        colc[pl.ds(g * LANES, LANES)] = zeros16i
        rowc[pl.ds(g * LANES, LANES)] = zeros16i
        return 0
    lax.fori_loop(0, EPT_PAD // LANES, prefill, 0)

    coff = cid * N_NODES

    for p in range(NPASS):
        lo = p * ROWS_PASS

        def zfill(r, _):
            for j in range(D_HALF // LANES):
                zbuf[r, pl.ds(j * LANES, LANES)] = zeros16f
            return 0
        lax.fori_loop(0, WB, zfill, 0)
        for b in range(RPT // WB):
            pltpu.sync_copy(
                zbuf, acc.at[pl.ds(sid * RPT + b * WB, WB), :])

        def vfill(g, _):
            valc[pl.ds(g * LANES, LANES)] = zeros16f
            return 0
        lax.fori_loop(0, EPT_PAD // LANES, vfill, 0)

        def compact(g, cnt):
            sl = pl.ds(g * LANES, LANES)
            rows = rowt[sl]
            mask = (rows >= lo) & (rows < lo + ROWS_PASS)
            plsc.store_compressed(colc.at[pl.ds(cnt, LANES)],
                                  colt[sl] + coff, mask=mask)
            plsc.store_compressed(rowc.at[pl.ds(cnt, LANES)],
                                  rows - lo, mask=mask)
            plsc.store_compressed(valc.at[pl.ds(cnt, LANES)],
                                  valt[sl], mask=mask)
            return cnt + plsc.all_reduce_population_count(mask)[0]
        cnt = lax.fori_loop(0, ngrp, compact, 0)
        plsc.subcore_barrier()

        nchunk = (cnt + EC - 1) // EC

        def fill_and_gather(c, cb, rb, ms):
            base = c * EC
            for k in range(EC // LANES):
                sl = pl.ds(base + k * LANES, LANES)
                dl = pl.ds(k * LANES, LANES)
                cb[dl] = colc[sl]
                rb[dl] = rowc[sl]
            pltpu.async_copy(pre_hbm.at[cb], ms, gsem)

        def scale_msgs(c, ms):
            base = c * EC

            def scale(g, _):
                vvals = valc[pl.ds(base + g * LANES, LANES)]
                for l in range(LANES):
                    v = vvals[l]
                    e = g * LANES + l
                    for j in range(D_HALF // LANES):
                        fl = pl.ds(j * LANES, LANES)
                        ms[e, fl] = ms[e, fl] * v
                return 0
            lax.fori_loop(0, EC // LANES, scale, 0)

        @pl.when(nchunk > 0)
        def _():
            fill_and_gather(0, cbuf, rbuf, msgs)

        def chunk(c, _):
            def stage(ms, rb, cb2, rb2, ms2):
                pltpu.make_async_copy(pre_hbm.at[cbuf], ms, gsem).wait()

                @pl.when(c + 1 < nchunk)
                def _():
                    @pl.when(c >= 1)
                    def _():
                        pltpu.make_async_copy(ms2, acc.at[rb2], ssem).wait()
                    fill_and_gather(c + 1, cb2, rb2, ms2)

                scale_msgs(c, ms)
                pltpu.async_copy(ms, acc.at[rb], ssem, add=True)

            @pl.when(c % 2 == 0)
            def _():
                stage(msgs, rbuf, cbuf2, rbuf2, msgs2)

            @pl.when(c % 2 == 1)
            def _():
                stage(msgs2, rbuf2, cbuf, rbuf, msgs)
            return 0
        lax.fori_loop(0, nchunk, chunk, 0)

        @pl.when(nchunk > 0)
        def _():
            pltpu.make_async_copy(msgs, acc.at[rbuf], ssem).wait()

        @pl.when(nchunk > 1)
        def _():
            pltpu.make_async_copy(msgs2, acc.at[rbuf2], ssem).wait()
        plsc.subcore_barrier()

        for b in range(RPT // WB):
            r0 = sid * RPT + b * WB
            pltpu.sync_copy(acc.at[pl.ds(r0, WB), :], zbuf)

            def relu_row(r, _):
                for j in range(D_HALF // LANES):
                    fl = pl.ds(j * LANES, LANES)
                    zbuf[r, fl] = jnp.maximum(zbuf[r, fl], 0.0)
                return 0
            lax.fori_loop(0, WB, relu_row, 0)
            pltpu.sync_copy(
                zbuf, out_hbm.at[cid, pl.ds(lo + r0, WB), :])


_sc_spmm = functools.partial(
    pl.kernel,
    mesh=plsc.VectorSubcoreMesh(core_axis_name="c", subcore_axis_name="s"),
    compiler_params=pltpu.CompilerParams(needs_layout_passes=False),
    out_type=jax.ShapeDtypeStruct((NUM_CORES, N_PAD, D_HALF), jnp.float32),
    scratch_types=[
        pltpu.VMEM((EPT,), jnp.int32),
        pltpu.VMEM((EPT,), jnp.int32),
        pltpu.VMEM((EPT,), jnp.float32),
        pltpu.VMEM((EPT_PAD,), jnp.int32),
        pltpu.VMEM((EPT_PAD,), jnp.int32),
        pltpu.VMEM((EPT_PAD,), jnp.float32),
        pltpu.VMEM((EC,), jnp.int32),
        pltpu.VMEM((EC,), jnp.int32),
        pltpu.VMEM((EC, D_HALF), jnp.float32),
        pltpu.VMEM((EC,), jnp.int32),
        pltpu.VMEM((EC,), jnp.int32),
        pltpu.VMEM((EC, D_HALF), jnp.float32),
        pltpu.VMEM((WB, D_HALF), jnp.float32),
        pltpu.VMEM_SHARED((ROWS_PASS, D_HALF), jnp.float32),
        pltpu.SemaphoreType.DMA,
        pltpu.SemaphoreType.DMA,
    ],
)(_sc_body)


def kernel(x, edge_index, edge_values, W):
    row = edge_index[0].astype(jnp.int32)
    col = edge_index[1].astype(jnp.int32)
    pre = _matmul_halves(x, W).reshape(NUM_CORES * N_NODES, D_HALF)
    out2 = _sc_spmm(pre, col, row, edge_values)
    return jnp.concatenate([out2[0, :N_NODES], out2[1, :N_NODES]], axis=1)


# speedup vs baseline: 3.0923x; 1.0082x over previous
import functools

import jax
import jax.numpy as jnp
from jax import lax
from jax.experimental import pallas as pl
from jax.experimental.pallas import tpu as pltpu
from jax.experimental.pallas import tpu_sc as plsc

N_NODES = 10000
N_EDGES = 160000
D_IN = 256
D_OUT = 256
D_HALF = D_OUT // 2

NUM_CORES = 2
NUM_SUBCORES = 16
LANES = 16

EPT = N_EDGES // NUM_SUBCORES
EC = 128
EPT_PAD = ((EPT + EC - 1) // EC) * EC
N_PAD = 10240
NPASS = 4
ROWS_PASS = N_PAD // NPASS
RPT = ROWS_PASS // NUM_SUBCORES
WB = 80


def _matmul_body(x_ref, w_ref, o_ref):
    o_ref[0, :, :] = jnp.dot(x_ref[...], w_ref[...],
                             preferred_element_type=jnp.float32)


def _matmul_halves(x, W):
    BR = 1000
    grid = (N_NODES // BR, NUM_CORES)
    return pl.pallas_call(
        _matmul_body,
        grid=grid,
        in_specs=[
            pl.BlockSpec((BR, D_IN), lambda i, j: (i, 0)),
            pl.BlockSpec((D_IN, D_HALF), lambda i, j: (0, j)),
        ],
        out_specs=pl.BlockSpec((1, BR, D_HALF), lambda i, j: (j, i, 0)),
        out_shape=jax.ShapeDtypeStruct((NUM_CORES, N_NODES, D_HALF),
                                       jnp.float32),
    )(x, W)


def _sc_body(pre_hbm, col_hbm, row_hbm, val_hbm, out_hbm,
             colt, rowt, valt, colc, rowc, valc,
             cbuf, rbuf, msgs, cbuf2, rbuf2, msgs2, acc, gsem, ssem):
    cid = lax.axis_index("c")
    sid = lax.axis_index("s")
    ebase = sid * EPT
    ngrp = EPT // LANES
    zeros16i = jnp.zeros((LANES,), jnp.int32)
    zeros16f = jnp.zeros((LANES,), jnp.float32)

    pltpu.sync_copy(col_hbm.at[pl.ds(ebase, EPT)], colt)
    pltpu.sync_copy(row_hbm.at[pl.ds(ebase, EPT)], rowt)
    pltpu.sync_copy(val_hbm.at[pl.ds(ebase, EPT)], valt)

    def prefill(g, _):
        colc[pl.ds(g * LANES, LANES)] = zeros16i
        rowc[pl.ds(g * LANES, LANES)] = zeros16i
        return 0
    lax.fori_loop(0, EPT_PAD // LANES, prefill, 0)

    coff = cid * N_NODES

    def run_pass(p, _):
        lo = p * ROWS_PASS

        def zfill(r, _):
            for j in range(D_HALF // LANES):
                msgs[r, pl.ds(j * LANES, LANES)] = zeros16f
            return 0
        lax.fori_loop(0, WB, zfill, 0)
        for b in range(RPT // WB):
            pltpu.sync_copy(
                msgs.at[pl.ds(0, WB), :],
                acc.at[pl.ds(sid * RPT + b * WB, WB), :])

        def vfill(g, _):
            valc[pl.ds(g * LANES, LANES)] = zeros16f
            return 0
        lax.fori_loop(0, EPT_PAD // LANES, vfill, 0)

        def compact(g, cnt):
            sl = pl.ds(g * LANES, LANES)
            rows = rowt[sl]
            mask = (rows >= lo) & (rows < lo + ROWS_PASS)
            plsc.store_compressed(colc.at[pl.ds(cnt, LANES)],
                                  colt[sl] + coff, mask=mask)
            plsc.store_compressed(rowc.at[pl.ds(cnt, LANES)],
                                  rows - lo, mask=mask)
            plsc.store_compressed(valc.at[pl.ds(cnt, LANES)],
                                  valt[sl], mask=mask)
            return cnt + plsc.all_reduce_population_count(mask)[0]
        cnt = lax.fori_loop(0, ngrp, compact, 0)
        plsc.subcore_barrier()

        nchunk = (cnt + EC - 1) // EC

        def fill_and_gather(c, cb, rb, ms):
            base = c * EC
            for k in range(EC // LANES):
                sl = pl.ds(base + k * LANES, LANES)
                dl = pl.ds(k * LANES, LANES)
                cb[dl] = colc[sl]
                rb[dl] = rowc[sl]
            pltpu.async_copy(pre_hbm.at[cb], ms, gsem)

        def scale_msgs(c, ms):
            base = c * EC

            def scale(g, _):
                vvals = valc[pl.ds(base + g * LANES, LANES)]
                for l in range(LANES):
                    v = vvals[l]
                    e = g * LANES + l
                    for j in range(D_HALF // LANES):
                        fl = pl.ds(j * LANES, LANES)
                        ms[e, fl] = ms[e, fl] * v
                return 0
            lax.fori_loop(0, EC // LANES, scale, 0)

        @pl.when(nchunk > 0)
        def _():
            fill_and_gather(0, cbuf, rbuf, msgs)

        def chunk(c, _):
            def stage(ms, rb, cb2, rb2, ms2):
                pltpu.make_async_copy(pre_hbm.at[cbuf], ms, gsem).wait()

                @pl.when(c + 1 < nchunk)
                def _():
                    @pl.when(c >= 1)
                    def _():
                        pltpu.make_async_copy(ms2, acc.at[rb2], ssem).wait()
                    fill_and_gather(c + 1, cb2, rb2, ms2)

                scale_msgs(c, ms)
                pltpu.async_copy(ms, acc.at[rb], ssem, add=True)

            @pl.when(c % 2 == 0)
            def _():
                stage(msgs, rbuf, cbuf2, rbuf2, msgs2)

            @pl.when(c % 2 == 1)
            def _():
                stage(msgs2, rbuf2, cbuf, rbuf, msgs)
            return 0
        lax.fori_loop(0, nchunk, chunk, 0)

        @pl.when(nchunk > 0)
        def _():
            pltpu.make_async_copy(msgs, acc.at[rbuf], ssem).wait()

        @pl.when(nchunk > 1)
        def _():
            pltpu.make_async_copy(msgs2, acc.at[rbuf2], ssem).wait()
        plsc.subcore_barrier()

        for b in range(RPT // WB):
            r0 = sid * RPT + b * WB
            pltpu.sync_copy(acc.at[pl.ds(r0, WB), :],
                            msgs.at[pl.ds(0, WB), :])

            def relu_row(r, _):
                for j in range(D_HALF // LANES):
                    fl = pl.ds(j * LANES, LANES)
                    msgs[r, fl] = jnp.maximum(msgs[r, fl], 0.0)
                return 0
            lax.fori_loop(0, WB, relu_row, 0)
            pltpu.sync_copy(
                msgs.at[pl.ds(0, WB), :],
                out_hbm.at[pl.ds(lo + r0, WB),
                           pl.ds(cid * D_HALF, D_HALF)])
        return 0
    lax.fori_loop(0, NPASS, run_pass, 0)


_sc_spmm = functools.partial(
    pl.kernel,
    mesh=plsc.VectorSubcoreMesh(core_axis_name="c", subcore_axis_name="s"),
    compiler_params=pltpu.CompilerParams(needs_layout_passes=False),
    out_type=jax.ShapeDtypeStruct((N_PAD, D_OUT), jnp.float32),
    scratch_types=[
        pltpu.VMEM((EPT,), jnp.int32),
        pltpu.VMEM((EPT,), jnp.int32),
        pltpu.VMEM((EPT,), jnp.float32),
        pltpu.VMEM((EPT_PAD,), jnp.int32),
        pltpu.VMEM((EPT_PAD,), jnp.int32),
        pltpu.VMEM((EPT_PAD,), jnp.float32),
        pltpu.VMEM((EC,), jnp.int32),
        pltpu.VMEM((EC,), jnp.int32),
        pltpu.VMEM((EC, D_HALF), jnp.float32),
        pltpu.VMEM((EC,), jnp.int32),
        pltpu.VMEM((EC,), jnp.int32),
        pltpu.VMEM((EC, D_HALF), jnp.float32),
        pltpu.VMEM_SHARED((ROWS_PASS, D_HALF), jnp.float32),
        pltpu.SemaphoreType.DMA,
        pltpu.SemaphoreType.DMA,
    ],
)(_sc_body)


def kernel(x, edge_index, edge_values, W):
    row = edge_index[0].astype(jnp.int32)
    col = edge_index[1].astype(jnp.int32)
    pre = _matmul_halves(x, W).reshape(NUM_CORES * N_NODES, D_HALF)
    out = _sc_spmm(pre, col, row, edge_values)
    return out[:N_NODES]


# speedup vs baseline: 3.1212x; 1.0093x over previous
import functools

import jax
import jax.numpy as jnp
from jax import lax
from jax.experimental import pallas as pl
from jax.experimental.pallas import tpu as pltpu
from jax.experimental.pallas import tpu_sc as plsc

N_NODES = 10000
N_EDGES = 160000
D_IN = 256
D_OUT = 256
D_HALF = D_OUT // 2

NUM_CORES = 2
NUM_SUBCORES = 16
LANES = 16

EPT = N_EDGES // NUM_SUBCORES
EC = 128
EPT_PAD = ((EPT + EC - 1) // EC) * EC
N_PAD = 10240
NPASS = 4
ROWS_PASS = N_PAD // NPASS
RPT = ROWS_PASS // NUM_SUBCORES
WB = 80


def _matmul_body(x_ref, w_ref, o_ref):
    o_ref[0, :, :] = jnp.dot(x_ref[...], w_ref[...],
                             preferred_element_type=jnp.float32)


def _matmul_halves(x, W):
    BR = 1000
    grid = (N_NODES // BR, NUM_CORES)
    return pl.pallas_call(
        _matmul_body,
        grid=grid,
        in_specs=[
            pl.BlockSpec((BR, D_IN), lambda i, j: (i, 0)),
            pl.BlockSpec((D_IN, D_HALF), lambda i, j: (0, j)),
        ],
        out_specs=pl.BlockSpec((1, BR, D_HALF), lambda i, j: (j, i, 0)),
        out_shape=jax.ShapeDtypeStruct((NUM_CORES, N_NODES, D_HALF),
                                       jnp.float32),
    )(x, W)


def _sc_body(pre_hbm, col_hbm, row_hbm, val_hbm, out_hbm,
             colt, rowt, valt, colc, rowc, valc,
             cbuf, rbuf, msgs, cbuf2, rbuf2, msgs2, acc, gsem, ssem):
    cid = lax.axis_index("c")
    sid = lax.axis_index("s")
    ebase = sid * EPT
    ngrp = EPT // LANES
    zeros16i = jnp.zeros((LANES,), jnp.int32)
    zeros16f = jnp.zeros((LANES,), jnp.float32)

    pltpu.sync_copy(col_hbm.at[pl.ds(ebase, EPT)], colt)
    pltpu.sync_copy(row_hbm.at[pl.ds(ebase, EPT)], rowt)
    pltpu.sync_copy(val_hbm.at[pl.ds(ebase, EPT)], valt)

    def prefill(g, _):
        colc[pl.ds(g * LANES, LANES)] = zeros16i
        rowc[pl.ds(g * LANES, LANES)] = zeros16i
        return 0
    lax.fori_loop(0, EPT_PAD // LANES, prefill, 0)

    coff = cid * N_NODES

    def run_pass(p, _):
        lo = p * ROWS_PASS

        def zfill(r, _):
            for j in range(D_HALF // LANES):
                msgs[r, pl.ds(j * LANES, LANES)] = zeros16f
            return 0
        lax.fori_loop(0, WB, zfill, 0)
        for b in range(RPT // WB):
            pltpu.sync_copy(
                msgs.at[pl.ds(0, WB), :],
                acc.at[pl.ds(sid * RPT + b * WB, WB), :])

        def vfill(g, _):
            valc[pl.ds(g * LANES, LANES)] = zeros16f
            return 0
        lax.fori_loop(0, EPT_PAD // LANES, vfill, 0)

        def compact(g, cnt):
            sl = pl.ds(g * LANES, LANES)
            rows = rowt[sl]
            mask = (rows >= lo) & (rows < lo + ROWS_PASS)
            plsc.store_compressed(colc.at[pl.ds(cnt, LANES)],
                                  colt[sl] + coff, mask=mask)
            plsc.store_compressed(rowc.at[pl.ds(cnt, LANES)],
                                  rows - lo, mask=mask)
            plsc.store_compressed(valc.at[pl.ds(cnt, LANES)],
                                  valt[sl], mask=mask)
            return cnt + plsc.all_reduce_population_count(mask)[0]
        cnt = lax.fori_loop(0, ngrp, compact, 0)
        plsc.subcore_barrier()

        nchunk = (cnt + EC - 1) // EC

        def fill_and_gather(c, cb, rb, ms):
            base = c * EC
            for k in range(EC // LANES):
                sl = pl.ds(base + k * LANES, LANES)
                dl = pl.ds(k * LANES, LANES)
                cb[dl] = colc[sl]
                rb[dl] = rowc[sl]
            pltpu.async_copy(pre_hbm.at[cb], ms, gsem)

        def scale_msgs(c, ms):
            base = c * EC

            def scale(g, _):
                vvals = valc[pl.ds(base + g * LANES, LANES)]
                for l in range(LANES):
                    v = vvals[l]
                    e = g * LANES + l
                    for j in range(D_HALF // LANES):
                        fl = pl.ds(j * LANES, LANES)
                        ms[e, fl] = ms[e, fl] * v
                return 0
            lax.fori_loop(0, EC // LANES, scale, 0)

        @pl.when(nchunk > 0)
        def _():
            fill_and_gather(0, cbuf, rbuf, msgs)

        def chunk(c, _):
            def stage(ms, rb, cb2, rb2, ms2):
                @pl.when(c + 1 < nchunk)
                def _():
                    @pl.when(c >= 1)
                    def _():
                        pltpu.make_async_copy(ms2, acc.at[rb2], ssem).wait()
                    fill_and_gather(c + 1, cb2, rb2, ms2)

                pltpu.make_async_copy(pre_hbm.at[cbuf], ms, gsem).wait()
                scale_msgs(c, ms)
                pltpu.async_copy(ms, acc.at[rb], ssem, add=True)

            @pl.when(c % 2 == 0)
            def _():
                stage(msgs, rbuf, cbuf2, rbuf2, msgs2)

            @pl.when(c % 2 == 1)
            def _():
                stage(msgs2, rbuf2, cbuf, rbuf, msgs)
            return 0
        lax.fori_loop(0, nchunk, chunk, 0)

        @pl.when(nchunk > 0)
        def _():
            pltpu.make_async_copy(msgs, acc.at[rbuf], ssem).wait()

        @pl.when(nchunk > 1)
        def _():
            pltpu.make_async_copy(msgs2, acc.at[rbuf2], ssem).wait()
        plsc.subcore_barrier()

        for b in range(RPT // WB):
            r0 = sid * RPT + b * WB
            pltpu.sync_copy(acc.at[pl.ds(r0, WB), :],
                            msgs.at[pl.ds(0, WB), :])

            def relu_row(r, _):
                for j in range(D_HALF // LANES):
                    fl = pl.ds(j * LANES, LANES)
                    msgs[r, fl] = jnp.maximum(msgs[r, fl], 0.0)
                return 0
            lax.fori_loop(0, WB, relu_row, 0)
            pltpu.sync_copy(
                msgs.at[pl.ds(0, WB), :],
                out_hbm.at[pl.ds(lo + r0, WB),
                           pl.ds(cid * D_HALF, D_HALF)])
        return 0
    lax.fori_loop(0, NPASS, run_pass, 0)


_sc_spmm = functools.partial(
    pl.kernel,
    mesh=plsc.VectorSubcoreMesh(core_axis_name="c", subcore_axis_name="s"),
    compiler_params=pltpu.CompilerParams(needs_layout_passes=False),
    out_type=jax.ShapeDtypeStruct((N_PAD, D_OUT), jnp.float32),
    scratch_types=[
        pltpu.VMEM((EPT,), jnp.int32),
        pltpu.VMEM((EPT,), jnp.int32),
        pltpu.VMEM((EPT,), jnp.float32),
        pltpu.VMEM((EPT_PAD,), jnp.int32),
        pltpu.VMEM((EPT_PAD,), jnp.int32),
        pltpu.VMEM((EPT_PAD,), jnp.float32),
        pltpu.VMEM((EC,), jnp.int32),
        pltpu.VMEM((EC,), jnp.int32),
        pltpu.VMEM((EC, D_HALF), jnp.float32),
        pltpu.VMEM((EC,), jnp.int32),
        pltpu.VMEM((EC,), jnp.int32),
        pltpu.VMEM((EC, D_HALF), jnp.float32),
        pltpu.VMEM_SHARED((ROWS_PASS, D_HALF), jnp.float32),
        pltpu.SemaphoreType.DMA,
        pltpu.SemaphoreType.DMA,
    ],
)(_sc_body)


def kernel(x, edge_index, edge_values, W):
    row = edge_index[0].astype(jnp.int32)
    col = edge_index[1].astype(jnp.int32)
    pre = _matmul_halves(x, W).reshape(NUM_CORES * N_NODES, D_HALF)
    out = _sc_spmm(pre, col, row, edge_values)
    return out[:N_NODES]
